# baseline (speedup 1.0000x reference)
import functools

import jax
import jax.numpy as jnp
from jax import lax
from jax.experimental import pallas as pl
from jax.experimental.pallas import tpu as pltpu
from jax.experimental.pallas import tpu_sc as plsc

N_NODES = 10000
N_EDGES = 320000
D = 128

NC = 2
NS = 16
NW = NC * NS
E_PER_W = N_EDGES // NW
CHUNK = 128
N_CHUNKS = 80
E_PAD_W = N_CHUNKS * CHUNK
GCH = 8
NGRP = N_CHUNKS // GCH
TRASH = N_NODES
ACC_ROWS = 10016
ROWS_PER_SUB = 624
TAIL_ROWS = N_NODES - ROWS_PER_SUB * NS


def _sc_aggregate(x, src_idx, dst_idx):
    mesh = plsc.VectorSubcoreMesh(core_axis_name="c", subcore_axis_name="s",
                                  num_cores=NC, num_subcores=NS)

    @functools.partial(
        pl.kernel,
        out_type=(
            jax.ShapeDtypeStruct((NC, N_NODES, D), jnp.float32),
            jax.ShapeDtypeStruct((NC, N_NODES, D), jnp.float32),
        ),
        mesh=mesh,
        scratch_types=[
            pltpu.VMEM((GCH, CHUNK), jnp.int32),
            pltpu.VMEM((GCH, CHUNK), jnp.int32),
            pltpu.VMEM((CHUNK, D), jnp.float32),
            pltpu.VMEM((CHUNK, D), jnp.float32),
            pltpu.VMEM_SHARED((ACC_ROWS, D), jnp.float32),
            pltpu.SemaphoreType.DMA,
            pltpu.SemaphoreType.DMA,
        ],
    )
    def k(x_hbm, src_hbm, dst_hbm, sum_hbm, cnt_hbm,
          src_v, dst_v, ga_v, gb_v, sh_acc, sem_a, sem_b):
        cid = lax.axis_index("c")
        sid = lax.axis_index("s")
        w = cid * NS + sid
        base = sid * ROWS_PER_SUB
        zero16 = jnp.zeros((16,), jnp.float32)
        one16 = jnp.ones((16,), jnp.float32)
        n_full = ROWS_PER_SUB // CHUNK
        rem = ROWS_PER_SUB - n_full * CHUNK

        def fill(buf, val16):
            @pl.loop(0, CHUNK)
            def _(r):
                @pl.loop(0, D // 16)
                def _(c):
                    buf[r, pl.ds(c * 16, 16)] = val16

        def zero_acc():
            for j in range(n_full):
                pltpu.sync_copy(gb_v,
                                sh_acc.at[pl.ds(base + j * CHUNK, CHUNK)])
            pltpu.sync_copy(gb_v.at[pl.ds(0, rem)],
                            sh_acc.at[pl.ds(base + n_full * CHUNK, rem)])

            @pl.when(sid == NS - 1)
            def _():
                tb = NS * ROWS_PER_SUB
                pltpu.sync_copy(gb_v.at[pl.ds(0, TAIL_ROWS + 16)],
                                sh_acc.at[pl.ds(tb, TAIL_ROWS + 16)])

        def copy_out(out_hbm):
            pltpu.sync_copy(sh_acc.at[pl.ds(base, ROWS_PER_SUB)],
                            out_hbm.at[cid].at[pl.ds(base, ROWS_PER_SUB)])

            @pl.when(sid == NS - 1)
            def _():
                tb = NS * ROWS_PER_SUB
                pltpu.sync_copy(sh_acc.at[pl.ds(tb, TAIL_ROWS)],
                                out_hbm.at[cid].at[pl.ds(tb, TAIL_ROWS)])

        fill(gb_v, zero16)
        zero_acc()
        fill(ga_v, one16)
        plsc.subcore_barrier()

        @pl.loop(0, NGRP)
        def _(g):
            pltpu.sync_copy(dst_hbm.at[w].at[pl.ds(g * GCH, GCH)], dst_v)
            cps = [pltpu.async_copy(ga_v, sh_acc.at[dst_v.at[p]],
                                    sem_a, add=True)
                   for p in range(GCH)]
            for cp in cps:
                cp.wait()

        plsc.subcore_barrier()
        copy_out(cnt_hbm)
        plsc.subcore_barrier()

        fill(gb_v, zero16)
        zero_acc()
        plsc.subcore_barrier()

        bufs = (ga_v, gb_v)
        sems = (sem_a, sem_b)

        @pl.loop(0, NGRP)
        def _(g):
            pltpu.sync_copy(src_hbm.at[w].at[pl.ds(g * GCH, GCH)], src_v)
            pltpu.sync_copy(dst_hbm.at[w].at[pl.ds(g * GCH, GCH)], dst_v)
            pend = pltpu.async_copy(x_hbm.at[src_v.at[0]], ga_v, sem_a)
            for p in range(GCH):
                if p + 1 < GCH:
                    nxt = pltpu.async_copy(x_hbm.at[src_v.at[p + 1]],
                                           bufs[(p + 1) % 2],
                                           sems[(p + 1) % 2])
                pend.wait()
                pltpu.sync_copy(bufs[p % 2], sh_acc.at[dst_v.at[p]],
                                add=True)
                if p + 1 < GCH:
                    pend = nxt

        plsc.subcore_barrier()
        copy_out(sum_hbm)

    return k(x, src_idx, dst_idx)


def _tc_finish_body(sum_ref, cnt_ref, x_ref, wl_ref, bl_ref, wr_ref,
                    g_ref, b_ref, o_ref):
    s = sum_ref[0] + sum_ref[1]
    n = cnt_ref[0, :, 0:1] + cnt_ref[1, :, 0:1]
    mean = s / jnp.maximum(n, 1.0)
    out = (jnp.dot(mean, wl_ref[...], preferred_element_type=jnp.float32)
           + jnp.dot(x_ref[...], wr_ref[...], preferred_element_type=jnp.float32)
           + bl_ref[...])
    mu = jnp.mean(out, axis=-1, keepdims=True)
    var = jnp.mean((out - mu) ** 2, axis=-1, keepdims=True)
    out = (out - mu) * lax.rsqrt(var + 1e-5)
    out = out * g_ref[...] + b_ref[...]
    o_ref[...] = jnp.maximum(out, 0.0)


def _tc_finish(sums, cnts, x, W_l, b_l, W_r, ln_gamma, ln_beta):
    R = 1000
    grid = (N_NODES // R,)
    full = lambda i: (0, 0)
    return pl.pallas_call(
        _tc_finish_body,
        grid=grid,
        in_specs=[
            pl.BlockSpec((NC, R, D), lambda i: (0, i, 0)),
            pl.BlockSpec((NC, R, D), lambda i: (0, i, 0)),
            pl.BlockSpec((R, D), lambda i: (i, 0)),
            pl.BlockSpec((D, D), full),
            pl.BlockSpec((1, D), full),
            pl.BlockSpec((D, D), full),
            pl.BlockSpec((1, D), full),
            pl.BlockSpec((1, D), full),
        ],
        out_specs=pl.BlockSpec((R, D), lambda i: (i, 0)),
        out_shape=jax.ShapeDtypeStruct((N_NODES, D), jnp.float32),
    )(sums, cnts, x, W_l, b_l.reshape(1, D), W_r,
      ln_gamma.reshape(1, D), ln_beta.reshape(1, D))


def kernel(x, edge_index, W_l, b_l, W_r, ln_gamma, ln_beta):
    ei = edge_index.astype(jnp.int32)
    pad = E_PAD_W - E_PER_W
    src = jnp.pad(ei[0].reshape(NW, E_PER_W), ((0, 0), (0, pad)),
                  constant_values=0).reshape(NW, N_CHUNKS, CHUNK)
    dst = jnp.pad(ei[1].reshape(NW, E_PER_W), ((0, 0), (0, pad)),
                  constant_values=TRASH).reshape(NW, N_CHUNKS, CHUNK)
    sums, cnts = _sc_aggregate(x, src, dst)
    return _tc_finish(sums, cnts, x, W_l, b_l, W_r, ln_gamma, ln_beta)


# speedup vs baseline: 2.2009x; 2.2009x over previous
import functools

import jax
import jax.numpy as jnp
from jax import lax
from jax.experimental import pallas as pl
from jax.experimental.pallas import tpu as pltpu
from jax.experimental.pallas import tpu_sc as plsc

N_NODES = 10000
N_EDGES = 320000
D = 128

NC = 2
NS = 16
NW = NC * NS
E_PER_W = N_EDGES // NW
CHUNK = 80
N_CHUNKS = E_PER_W // CHUNK
GCH = 25
NGRP = N_CHUNKS // GCH
ROWS_PER_SUB = 624
TAIL_ROWS = N_NODES - ROWS_PER_SUB * NS


def _sc_aggregate(x, src_idx, dst_idx):
    mesh = plsc.VectorSubcoreMesh(core_axis_name="c", subcore_axis_name="s",
                                  num_cores=NC, num_subcores=NS)

    @functools.partial(
        pl.kernel,
        out_type=(
            jax.ShapeDtypeStruct((NC, N_NODES, D), jnp.float32),
            jax.ShapeDtypeStruct((NC, N_NODES, D), jnp.float32),
        ),
        mesh=mesh,
        scratch_types=[
            pltpu.VMEM((GCH, CHUNK), jnp.int32),
            pltpu.VMEM((GCH, CHUNK), jnp.int32),
            pltpu.VMEM((CHUNK, D), jnp.float32),
            pltpu.VMEM((CHUNK, D), jnp.float32),
            pltpu.VMEM((CHUNK, D), jnp.float32),
            pltpu.VMEM_SHARED((N_NODES, D), jnp.float32),
            pltpu.SemaphoreType.DMA,
            pltpu.SemaphoreType.DMA,
        ],
    )
    def k(x_hbm, src_hbm, dst_hbm, sum_hbm, cnt_hbm,
          src_v, dst_v, rows_v, ga_v, gb_v, sh_acc, sem_a, sem_b):
        cid = lax.axis_index("c")
        sid = lax.axis_index("s")
        w = cid * NS + sid
        base = sid * ROWS_PER_SUB
        zero16 = jnp.zeros((16,), jnp.float32)
        one16 = jnp.ones((16,), jnp.float32)
        n_full = ROWS_PER_SUB // CHUNK
        rem = ROWS_PER_SUB - n_full * CHUNK

        def fill_rows(val16):
            @pl.loop(0, CHUNK)
            def _(r):
                @pl.loop(0, D // 16)
                def _(c):
                    rows_v[r, pl.ds(c * 16, 16)] = val16

        def zero_acc():
            for j in range(n_full):
                pltpu.sync_copy(rows_v,
                                sh_acc.at[pl.ds(base + j * CHUNK, CHUNK)])
            pltpu.sync_copy(rows_v.at[pl.ds(0, rem)],
                            sh_acc.at[pl.ds(base + n_full * CHUNK, rem)])

            @pl.when(sid == NS - 1)
            def _():
                tb = NS * ROWS_PER_SUB
                pltpu.sync_copy(rows_v.at[pl.ds(0, TAIL_ROWS)],
                                sh_acc.at[pl.ds(tb, TAIL_ROWS)])

        def copy_out(out_hbm):
            pltpu.sync_copy(sh_acc.at[pl.ds(base, ROWS_PER_SUB)],
                            out_hbm.at[cid].at[pl.ds(base, ROWS_PER_SUB)])

            @pl.when(sid == NS - 1)
            def _():
                tb = NS * ROWS_PER_SUB
                pltpu.sync_copy(sh_acc.at[pl.ds(tb, TAIL_ROWS)],
                                out_hbm.at[cid].at[pl.ds(tb, TAIL_ROWS)])

        fill_rows(zero16)
        zero_acc()
        fill_rows(one16)
        plsc.subcore_barrier()

        @pl.loop(0, NGRP)
        def _(g):
            pltpu.sync_copy(dst_hbm.at[w * NGRP + g], dst_v)
            cps = [pltpu.async_copy(rows_v, sh_acc.at[dst_v.at[p]],
                                    sem_a, add=True)
                   for p in range(GCH)]
            for cp in cps:
                cp.wait()

        plsc.subcore_barrier()
        copy_out(cnt_hbm)
        plsc.subcore_barrier()

        fill_rows(zero16)
        zero_acc()
        plsc.subcore_barrier()

        bufs = (ga_v, gb_v)
        sems = (sem_a, sem_b)

        @pl.loop(0, NGRP)
        def _(g):
            pltpu.sync_copy(src_hbm.at[w * NGRP + g], src_v)
            pltpu.sync_copy(dst_hbm.at[w * NGRP + g], dst_v)
            pend = pltpu.async_copy(x_hbm.at[src_v.at[0]], ga_v, sem_a)
            for p in range(GCH):
                if p + 1 < GCH:
                    nxt = pltpu.async_copy(x_hbm.at[src_v.at[p + 1]],
                                           bufs[(p + 1) % 2],
                                           sems[(p + 1) % 2])
                pend.wait()
                pltpu.sync_copy(bufs[p % 2], sh_acc.at[dst_v.at[p]],
                                add=True)
                if p + 1 < GCH:
                    pend = nxt

        plsc.subcore_barrier()
        copy_out(sum_hbm)

    return k(x, src_idx, dst_idx)


def _tc_finish_body(sum_ref, cnt_ref, x_ref, wl_ref, bl_ref, wr_ref,
                    g_ref, b_ref, o_ref):
    s = sum_ref[0] + sum_ref[1]
    n = cnt_ref[0, :, 0:1] + cnt_ref[1, :, 0:1]
    mean = s / jnp.maximum(n, 1.0)
    out = (jnp.dot(mean, wl_ref[...], preferred_element_type=jnp.float32)
           + jnp.dot(x_ref[...], wr_ref[...], preferred_element_type=jnp.float32)
           + bl_ref[...])
    mu = jnp.mean(out, axis=-1, keepdims=True)
    var = jnp.mean((out - mu) ** 2, axis=-1, keepdims=True)
    out = (out - mu) * lax.rsqrt(var + 1e-5)
    out = out * g_ref[...] + b_ref[...]
    o_ref[...] = jnp.maximum(out, 0.0)


def _tc_finish(sums, cnts, x, W_l, b_l, W_r, ln_gamma, ln_beta):
    R = 1000
    grid = (N_NODES // R,)
    full = lambda i: (0, 0)
    return pl.pallas_call(
        _tc_finish_body,
        grid=grid,
        in_specs=[
            pl.BlockSpec((NC, R, D), lambda i: (0, i, 0)),
            pl.BlockSpec((NC, R, D), lambda i: (0, i, 0)),
            pl.BlockSpec((R, D), lambda i: (i, 0)),
            pl.BlockSpec((D, D), full),
            pl.BlockSpec((1, D), full),
            pl.BlockSpec((D, D), full),
            pl.BlockSpec((1, D), full),
            pl.BlockSpec((1, D), full),
        ],
        out_specs=pl.BlockSpec((R, D), lambda i: (i, 0)),
        out_shape=jax.ShapeDtypeStruct((N_NODES, D), jnp.float32),
    )(sums, cnts, x, W_l, b_l.reshape(1, D), W_r,
      ln_gamma.reshape(1, D), ln_beta.reshape(1, D))


def kernel(x, edge_index, W_l, b_l, W_r, ln_gamma, ln_beta):
    ei = edge_index.astype(jnp.int32)
    src = ei[0].reshape(NW * NGRP, GCH, CHUNK)
    dst = ei[1].reshape(NW * NGRP, GCH, CHUNK)
    sums, cnts = _sc_aggregate(x, src, dst)
    return _tc_finish(sums, cnts, x, W_l, b_l, W_r, ln_gamma, ln_beta)
